# baseline (speedup 1.0000x reference)
import jax
import jax.numpy as jnp
from jax import lax
from jax.experimental import pallas as pl
from jax.experimental.pallas import tpu as pltpu
from jax.experimental.pallas import tpu_sc as plsc

B = 4096
H = 50
D = 128
C = 10
LANES = 16
DCH = D // LANES

NC = 2
NS = 16
NW = NC * NS

G = 2
RPS = G * H
SPW = B // G // NW
BPW = B // NW
NBUF = 4
NGRP = SPW // NBUF


def _fused_body(table_hbm, textg_hbm, w_hbm, b_hbm, out_hbm,
                idx_v, rows_v, out_v, w_v, b_v, *sems):
    wid = lax.axis_index("s") * NC + lax.axis_index("c")
    g0 = wid * SPW
    pltpu.sync_copy(w_hbm, w_v)
    pltpu.sync_copy(b_hbm, b_v)
    pltpu.sync_copy(textg_hbm.at[pl.ds(g0, SPW)], idx_v)

    def start(i, s):
        pltpu.make_async_copy(
            table_hbm.at[idx_v.at[i]], rows_v.at[s], sems[s]).start()

    def wait(s):
        pltpu.make_async_copy(
            table_hbm.at[idx_v.at[0]], rows_v.at[s], sems[s]).wait()

    for s in range(NBUF):
        start(s, s)

    def group(gidx, carry):
        for s in range(NBUF):
            i = gidx * NBUF + s
            wait(s)
            for e in range(G):
                def body(l, accs, _e=e):
                    r = _e * H + l
                    return tuple(accs[c] + rows_v[s, r, pl.ds(c * LANES, LANES)]
                                 for c in range(DCH))
                accs = lax.fori_loop(
                    0, H, body,
                    tuple(jnp.zeros((LANES,), jnp.float32) for _ in range(DCH)),
                    unroll=5)
                row_out = i * G + e
                lane = lax.iota(jnp.int32, LANES)
                last = jnp.full((LANES,), LANES - 1, jnp.int32)
                bvec = b_v[...]
                vec = bvec
                for c in range(C):
                    p = accs[0] * w_v[c, pl.ds(0, LANES)]
                    for k in range(1, DCH):
                        p = p + accs[k] * w_v[c, pl.ds(k * LANES, LANES)]
                    tot = p
                    for sh in (8, 4, 2, 1):
                        tot = tot + lax.gather(
                            tot, (lane ^ sh)[:, None],
                            dimension_numbers=lax.GatherDimensionNumbers(
                                offset_dims=(), collapsed_slice_dims=(0,),
                                start_index_map=(0,)),
                            slice_sizes=(1,),
                            mode=lax.GatherScatterMode.PROMISE_IN_BOUNDS)
                    vec = jnp.where(lane == c, tot + bvec, vec)
                out_v[row_out, :] = vec

            nxt = i + NBUF

            @pl.when(nxt < SPW)
            def _():
                start(nxt, s)
        return carry

    lax.fori_loop(0, NGRP, group, 0)
    pltpu.sync_copy(out_v, out_hbm.at[pl.ds(wid * BPW, BPW)])


_fused = pl.kernel(
    _fused_body,
    out_type=jax.ShapeDtypeStruct((B, LANES), jnp.float32),
    mesh=plsc.VectorSubcoreMesh(core_axis_name="c", subcore_axis_name="s"),
    scratch_types=[
        pltpu.VMEM((SPW, RPS), jnp.int32),
        pltpu.VMEM((NBUF, RPS, D), jnp.float32),
        pltpu.VMEM((BPW, LANES), jnp.float32),
        pltpu.VMEM((C, D), jnp.float32),
        pltpu.VMEM((LANES,), jnp.float32),
    ] + [pltpu.SemaphoreType.DMA] * NBUF,
)


def kernel(text, emb_table, fc_w, fc_b):
    textg = text.astype(jnp.int32).reshape(B // G, RPS)
    wt = fc_w * jnp.float32(1.0 / H)
    bias = jnp.zeros((LANES,), jnp.float32).at[:C].set(fc_b)
    return _fused(emb_table, textg, wt, bias)[:, :C]


# speedup vs baseline: 1.2196x; 1.2196x over previous
import jax
import jax.numpy as jnp
from jax import lax
from jax.experimental import pallas as pl
from jax.experimental.pallas import tpu as pltpu
from jax.experimental.pallas import tpu_sc as plsc

B = 4096
H = 50
D = 128
C = 10
LANES = 16
DCH = D // LANES

NC = 2
NS = 16
NW = NC * NS

G = 2
RPS = G * H
SPW = B // G // NW
BPW = B // NW
NBUF = 4
NGRP = SPW // NBUF


def _pool_body(table_hbm, textg_hbm, out_hbm, idx_v, rows_v, out_v, *sems):
    wid = lax.axis_index("s") * NC + lax.axis_index("c")
    g0 = wid * SPW
    pltpu.sync_copy(textg_hbm.at[pl.ds(g0, SPW)], idx_v)

    def start(i, s):
        pltpu.make_async_copy(
            table_hbm.at[idx_v.at[i]], rows_v.at[s], sems[s]).start()

    def wait(s):
        pltpu.make_async_copy(
            table_hbm.at[idx_v.at[0]], rows_v.at[s], sems[s]).wait()

    for s in range(NBUF):
        start(s, s)

    def group(gidx, carry):
        for s in range(NBUF):
            i = gidx * NBUF + s
            wait(s)
            for e in range(G):
                def body(l, accs, _e=e):
                    r = _e * H + l
                    return tuple(accs[c] + rows_v[s, r, pl.ds(c * LANES, LANES)]
                                 for c in range(DCH))
                accs = lax.fori_loop(
                    0, H, body,
                    tuple(jnp.zeros((LANES,), jnp.float32) for _ in range(DCH)),
                    unroll=5)
                row_out = i * G + e
                for c in range(DCH):
                    out_v[row_out, pl.ds(c * LANES, LANES)] = accs[c]

            nxt = i + NBUF

            @pl.when(nxt < SPW)
            def _():
                start(nxt, s)
        return carry

    lax.fori_loop(0, NGRP, group, 0)
    pltpu.sync_copy(out_v, out_hbm.at[pl.ds(wid * BPW, BPW)])


_pool = pl.kernel(
    _pool_body,
    out_type=jax.ShapeDtypeStruct((B, D), jnp.float32),
    mesh=plsc.VectorSubcoreMesh(core_axis_name="c", subcore_axis_name="s"),
    scratch_types=[
        pltpu.VMEM((SPW, RPS), jnp.int32),
        pltpu.VMEM((NBUF, RPS, D), jnp.float32),
        pltpu.VMEM((BPW, D), jnp.float32),
    ] + [pltpu.SemaphoreType.DMA] * NBUF,
)


def _fc_body(x_ref, w_ref, b_ref, o_ref):
    o_ref[...] = jnp.dot(x_ref[...], w_ref[...],
                         preferred_element_type=jnp.float32) + b_ref[...]


def kernel(text, emb_table, fc_w, fc_b):
    textg = text.astype(jnp.int32).reshape(B // G, RPS)
    pooled = _pool(emb_table, textg)
    wt = fc_w.T * jnp.float32(1.0 / H)
    out = pl.pallas_call(
        _fc_body,
        out_shape=jax.ShapeDtypeStruct((B, C), jnp.float32),
    )(pooled, wt, fc_b.reshape(1, C))
    return out
